# baseline (speedup 1.0000x reference)
import functools

import jax
import jax.numpy as jnp
from jax import lax
from jax.experimental import pallas as pl
from jax.experimental.pallas import tpu as pltpu
from jax.experimental.pallas import tpu_sc as plsc

BATCH = 4096
MAX_POS = 200
EMBED_DIM = 64
EPS = 1e-12

_NUM_WORKERS = 32
_B_PER_W = BATCH // _NUM_WORKERS


_GATHER_DN = lax.GatherDimensionNumbers(
    offset_dims=(), collapsed_slice_dims=(0,), start_index_map=(0,))


def _lane_sum(v):
    lanes = lax.iota(jnp.int32, 16)
    for sh in (1, 2, 4, 8):
        idx = (lanes ^ sh)[:, None]
        v = v + lax.gather(v, idx, _GATHER_DN, slice_sizes=(1,),
                           mode=lax.GatherScatterMode.PROMISE_IN_BOUNDS)
    return v


def _rsqrt16(v):
    y = lax.bitcast_convert_type(
        jnp.full((16,), 0x5F3759DF, dtype=jnp.int32)
        - lax.shift_right_arithmetic(lax.bitcast_convert_type(v, jnp.int32), 1),
        jnp.float32,
    )
    for _ in range(3):
        y = y * (1.5 - 0.5 * v * y * y)
    return y


def _body(ids_hbm, tok_hbm, pos_hbm, gam_hbm, bet_hbm, out_hbm,
          pos_v, idx_v, rows_v, par_v, sem):
    wid = lax.axis_index("s") * 2 + lax.axis_index("c")

    pltpu.sync_copy(pos_hbm, pos_v)
    pltpu.sync_copy(gam_hbm, par_v.at[0])
    pltpu.sync_copy(bet_hbm, par_v.at[1])
    gam = [par_v[0, pl.ds(16 * c, 16)] for c in range(4)]
    bet = [par_v[1, pl.ds(16 * c, 16)] for c in range(4)]

    def chunk_body(i, carry):
        b = wid * _B_PER_W + i
        pltpu.sync_copy(ids_hbm.at[b], idx_v)
        cp0 = pltpu.async_copy(
            tok_hbm.at[idx_v.at[pl.ds(0, 128)]], rows_v.at[pl.ds(0, 128)], sem)
        cp1 = pltpu.async_copy(
            tok_hbm.at[idx_v.at[pl.ds(128, 72)]], rows_v.at[pl.ds(128, 72)], sem)
        cp0.wait()
        cp1.wait()

        def row_body(r, c2):
            e = [rows_v[r, pl.ds(16 * c, 16)] + pos_v[r, pl.ds(16 * c, 16)]
                 for c in range(4)]
            s = (e[0] + e[1]) + (e[2] + e[3])
            q = (e[0] * e[0] + e[1] * e[1]) + (e[2] * e[2] + e[3] * e[3])
            mean = _lane_sum(s) * (1.0 / EMBED_DIM)
            var = _lane_sum(q) * (1.0 / EMBED_DIM) - mean * mean + EPS
            rstd = _rsqrt16(var)
            for c in range(4):
                rows_v[r, pl.ds(16 * c, 16)] = (e[c] - mean) * rstd * gam[c] + bet[c]
            return c2

        lax.fori_loop(0, MAX_POS, row_body, 0, unroll=4)
        pltpu.sync_copy(rows_v, out_hbm.at[b])
        return carry

    lax.fori_loop(0, _B_PER_W, chunk_body, 0)


@functools.partial(
    pl.kernel,
    mesh=plsc.VectorSubcoreMesh(core_axis_name="c", subcore_axis_name="s"),
    out_type=jax.ShapeDtypeStruct((BATCH, MAX_POS, EMBED_DIM), jnp.float32),
    compiler_params=pltpu.CompilerParams(use_tc_tiling_on_sc=False),
    scratch_types=[
        pltpu.VMEM((MAX_POS, EMBED_DIM), jnp.float32),
        pltpu.VMEM((MAX_POS,), jnp.int32),
        pltpu.VMEM((MAX_POS, EMBED_DIM), jnp.float32),
        pltpu.VMEM((2, EMBED_DIM), jnp.float32),
        pltpu.SemaphoreType.DMA,
    ],
)
def _embed_ln_sc(ids_hbm, tok_hbm, pos_hbm, gam_hbm, bet_hbm, out_hbm,
                 pos_v, idx_v, rows_v, par_v, sem):
    _body(ids_hbm, tok_hbm, pos_hbm, gam_hbm, bet_hbm, out_hbm,
          pos_v, idx_v, rows_v, par_v, sem)


def kernel(input_ids, token_table, pos_table, gamma, beta):
    ids = input_ids.astype(jnp.int32)
    return _embed_ln_sc(ids, token_table, pos_table, gamma, beta)


# speedup vs baseline: 1.0285x; 1.0285x over previous
import functools

import jax
import jax.numpy as jnp
from jax import lax
from jax.experimental import pallas as pl
from jax.experimental.pallas import tpu as pltpu
from jax.experimental.pallas import tpu_sc as plsc

BATCH = 4096
MAX_POS = 200
EMBED_DIM = 64
EPS = 1e-12

_NUM_WORKERS = 32
_ROWS = BATCH * MAX_POS
_RPW = _ROWS // _NUM_WORKERS
_CHUNK = 2 * MAX_POS
_NCHUNK = _RPW // _CHUNK
_NBUF = 4
_GROUPS = _CHUNK // 16


_GATHER_DN = lax.GatherDimensionNumbers(
    offset_dims=(), collapsed_slice_dims=(0,), start_index_map=(0,))


def _lane_sum(v):
    lanes = lax.iota(jnp.int32, 16)
    for sh in (1, 2, 4, 8):
        idx = (lanes ^ sh)[:, None]
        v = v + lax.gather(v, idx, _GATHER_DN, slice_sizes=(1,),
                           mode=lax.GatherScatterMode.PROMISE_IN_BOUNDS)
    return v


def _rsqrt16(v):
    y = lax.bitcast_convert_type(
        jnp.full((16,), 0x5F3759DF, dtype=jnp.int32)
        - lax.shift_right_arithmetic(lax.bitcast_convert_type(v, jnp.int32), 1),
        jnp.float32,
    )
    for _ in range(3):
        y = y * (1.5 - 0.5 * v * y * y)
    return y


def _body(ids_hbm, tok_hbm, pos_hbm, gam_hbm, bet_hbm, out_hbm,
          pos_v, par_v, stats_v, mrs_v, idx_bufs, row_bufs,
          isems, gsems, osems):
    wid = lax.axis_index("s") * 2 + lax.axis_index("c")
    base = wid * _RPW

    pltpu.sync_copy(pos_hbm, pos_v)
    pltpu.sync_copy(gam_hbm, par_v.at[0])
    pltpu.sync_copy(bet_hbm, par_v.at[1])
    gam = [par_v[0, pl.ds(16 * c, 16)] for c in range(4)]
    bet = [par_v[1, pl.ds(16 * c, 16)] for c in range(4)]

    lanes = lax.iota(jnp.int32, 16)
    colb = lanes * 16

    def chunk_off(i):
        return pl.multiple_of(base + i * _CHUNK, 8)

    def start_idx_copy(i, b):
        pltpu.async_copy(ids_hbm.at[pl.ds(chunk_off(i), _CHUNK)],
                         idx_bufs[b], isems[b])

    def wait_idx_copy(b):
        pltpu.make_async_copy(ids_hbm.at[pl.ds(0, _CHUNK)],
                              idx_bufs[b], isems[b]).wait()

    def start_gather(i, b):
        for j in range(4):
            off = pl.multiple_of(128 * j, 8)
            n = min(128, _CHUNK - 128 * j)
            pltpu.async_copy(tok_hbm.at[idx_bufs[b].at[pl.ds(off, n)]],
                             row_bufs[b].at[pl.ds(off, n)], gsems[b])

    def wait_gather(b):
        pltpu.make_async_copy(out_hbm.at[pl.ds(0, _CHUNK)],
                              row_bufs[b], gsems[b]).wait()

    def start_out_copy(i, b):
        pltpu.async_copy(row_bufs[b],
                         out_hbm.at[pl.ds(chunk_off(i), _CHUNK)], osems[b])

    def wait_out_copy(b):
        pltpu.make_async_copy(out_hbm.at[pl.ds(0, _CHUNK)],
                              row_bufs[b], osems[b]).wait()

    def compute_chunk(rows_b):
        def row_body_v1(row, carry):
            prow = row - jnp.where(row >= MAX_POS, MAX_POS, 0)
            e = []
            for c in range(4):
                x = rows_b[row, pl.ds(16 * c, 16)]
                p = pos_v[prow, pl.ds(16 * c, 16)]
                e.append(x + p)
            s = (e[0] + e[1]) + (e[2] + e[3])
            q = (e[0] * e[0] + e[1] * e[1]) + (e[2] * e[2] + e[3] * e[3])
            mean = _lane_sum(s) * (1.0 / EMBED_DIM)
            var = _lane_sum(q) * (1.0 / EMBED_DIM) - mean * mean + EPS
            rstd = _rsqrt16(var)
            for c in range(4):
                rows_b[row, pl.ds(16 * c, 16)] = (e[c] - mean) * rstd * gam[c] + bet[c]
            return carry

        lax.fori_loop(0, _CHUNK, row_body_v1, 0, unroll=4)
        return

        def group_body(g, carry):
            rbase = g * 16
            for rr in range(16):
                row = rbase + rr
                prow = row - jnp.where(row >= MAX_POS, MAX_POS, 0)
                e = []
                for c in range(4):
                    x = rows_b[row, pl.ds(16 * c, 16)]
                    p = pos_v[prow, pl.ds(16 * c, 16)]
                    e.append(x + p)
                    rows_b[row, pl.ds(16 * c, 16)] = e[c]
                s = (e[0] + e[1]) + (e[2] + e[3])
                q = (e[0] * e[0] + e[1] * e[1]) + (e[2] * e[2] + e[3] * e[3])
                stats_v[pl.ds(16 * rr, 16)] = s
                stats_v[pl.ds(256 + 16 * rr, 16)] = q
            accs = plsc.load_gather(stats_v, [colb])
            accq = plsc.load_gather(stats_v, [colb + 256])
            for d in range(1, 16):
                idx = colb + d
                accs = accs + plsc.load_gather(stats_v, [idx])
                accq = accq + plsc.load_gather(stats_v, [idx + 256])
            mean = accs * (1.0 / EMBED_DIM)
            var = accq * (1.0 / EMBED_DIM) - mean * mean + EPS
            rstd = _rsqrt16(var)
            mrs_v[pl.ds(0, 16)] = mean
            mrs_v[pl.ds(16, 16)] = rstd
            for rr in range(16):
                row = rbase + rr
                m = plsc.load_gather(mrs_v, [jnp.full((16,), rr, jnp.int32)])
                r = plsc.load_gather(mrs_v, [jnp.full((16,), 16 + rr, jnp.int32)])
                for c in range(4):
                    ec = rows_b[row, pl.ds(16 * c, 16)]
                    rows_b[row, pl.ds(16 * c, 16)] = (ec - m) * r * gam[c] + bet[c]
            return carry

        lax.fori_loop(0, _GROUPS, group_body, 0)

    def outer_body(go, carry):
        for b in range(_NBUF):
            i = go * _NBUF + b
            start_idx_copy(i, b)
            wait_idx_copy(b)
            start_gather(i, b)
            wait_gather(b)
            compute_chunk(row_bufs[b])
            start_out_copy(i, b)
            wait_out_copy(b)
        return carry

    lax.fori_loop(0, _NCHUNK // _NBUF, outer_body, 0)


@functools.partial(
    pl.kernel,
    mesh=plsc.VectorSubcoreMesh(core_axis_name="c", subcore_axis_name="s"),
    out_type=jax.ShapeDtypeStruct((_ROWS, EMBED_DIM), jnp.float32),
    compiler_params=pltpu.CompilerParams(
        use_tc_tiling_on_sc=False, needs_layout_passes=False),
    scratch_types=(
        [
            pltpu.VMEM((MAX_POS, EMBED_DIM), jnp.float32),
            pltpu.VMEM((2, EMBED_DIM), jnp.float32),
            pltpu.VMEM((2 * 16 * 16,), jnp.float32),
            pltpu.VMEM((32,), jnp.float32),
        ]
        + [pltpu.VMEM((_CHUNK,), jnp.int32) for _ in range(_NBUF)]
        + [pltpu.VMEM((_CHUNK, EMBED_DIM), jnp.float32) for _ in range(_NBUF)]
        + [pltpu.SemaphoreType.DMA for _ in range(3 * _NBUF)]
    ),
)
def _embed_ln_sc(ids_hbm, tok_hbm, pos_hbm, gam_hbm, bet_hbm, out_hbm,
                 pos_v, par_v, stats_v, mrs_v,
                 i0, i1, i2, i3, r0, r1, r2, r3,
                 s0, s1, s2, s3, s4, s5, s6, s7, s8, s9, s10, s11):
    _body(ids_hbm, tok_hbm, pos_hbm, gam_hbm, bet_hbm, out_hbm,
          pos_v, par_v, stats_v, mrs_v,
          [i0, i1, i2, i3], [r0, r1, r2, r3],
          [s0, s1, s2, s3], [s4, s5, s6, s7], [s8, s9, s10, s11])


def kernel(input_ids, token_table, pos_table, gamma, beta):
    ids = input_ids.astype(jnp.int32).reshape(-1)
    out = _embed_ln_sc(ids, token_table, pos_table, gamma, beta)
    return out.reshape(BATCH, MAX_POS, EMBED_DIM)


# speedup vs baseline: 1.1288x; 1.0975x over previous
import functools

import jax
import jax.numpy as jnp
from jax import lax
from jax.experimental import pallas as pl
from jax.experimental.pallas import tpu as pltpu
from jax.experimental.pallas import tpu_sc as plsc

BATCH = 4096
MAX_POS = 200
EMBED_DIM = 64
EPS = 1e-12

_NUM_WORKERS = 32
_ROWS = BATCH * MAX_POS
_RPW = _ROWS // _NUM_WORKERS
_CHUNK = 2 * MAX_POS
_NCHUNK = _RPW // _CHUNK
_NBUF = 4
_GROUPS = _CHUNK // 16


_GATHER_DN = lax.GatherDimensionNumbers(
    offset_dims=(), collapsed_slice_dims=(0,), start_index_map=(0,))


def _lane_sum(v):
    lanes = lax.iota(jnp.int32, 16)
    for sh in (1, 2, 4, 8):
        idx = (lanes ^ sh)[:, None]
        v = v + lax.gather(v, idx, _GATHER_DN, slice_sizes=(1,),
                           mode=lax.GatherScatterMode.PROMISE_IN_BOUNDS)
    return v


def _rsqrt16(v):
    y = lax.bitcast_convert_type(
        jnp.full((16,), 0x5F3759DF, dtype=jnp.int32)
        - lax.shift_right_arithmetic(lax.bitcast_convert_type(v, jnp.int32), 1),
        jnp.float32,
    )
    for _ in range(3):
        y = y * (1.5 - 0.5 * v * y * y)
    return y


def _body(ids_hbm, tok_hbm, pos_hbm, gam_hbm, bet_hbm, out_hbm,
          pos_v, par_v, stats_v, mrs_v, idx_bufs, row_bufs,
          isems, gsems, osems):
    wid = lax.axis_index("s") * 2 + lax.axis_index("c")
    base = wid * _RPW

    pltpu.sync_copy(pos_hbm, pos_v)
    pltpu.sync_copy(gam_hbm, par_v.at[0])
    pltpu.sync_copy(bet_hbm, par_v.at[1])
    gam = [par_v[0, pl.ds(16 * c, 16)] for c in range(4)]
    bet = [par_v[1, pl.ds(16 * c, 16)] for c in range(4)]

    lanes = lax.iota(jnp.int32, 16)
    colb = lanes * 16

    def chunk_off(i):
        return pl.multiple_of(base + i * _CHUNK, 8)

    def start_idx_copy(i, b):
        pltpu.async_copy(ids_hbm.at[pl.ds(chunk_off(i), _CHUNK)],
                         idx_bufs[b], isems[b])

    def wait_idx_copy(b):
        pltpu.make_async_copy(ids_hbm.at[pl.ds(0, _CHUNK)],
                              idx_bufs[b], isems[b]).wait()

    def start_gather(i, b):
        for j in range(4):
            off = pl.multiple_of(128 * j, 8)
            n = min(128, _CHUNK - 128 * j)
            pltpu.async_copy(tok_hbm.at[idx_bufs[b].at[pl.ds(off, n)]],
                             row_bufs[b].at[pl.ds(off, n)], gsems[b])

    def wait_gather(b):
        pltpu.make_async_copy(out_hbm.at[pl.ds(0, _CHUNK)],
                              row_bufs[b], gsems[b]).wait()

    def start_out_copy(i, b):
        pltpu.async_copy(row_bufs[b],
                         out_hbm.at[pl.ds(chunk_off(i), _CHUNK)], osems[b])

    def wait_out_copy(b):
        pltpu.make_async_copy(out_hbm.at[pl.ds(0, _CHUNK)],
                              row_bufs[b], osems[b]).wait()

    def compute_chunk(rows_b):
        def row_body_v1(row, carry):
            prow = row - jnp.where(row >= MAX_POS, MAX_POS, 0)
            e = []
            for c in range(4):
                x = rows_b[row, pl.ds(16 * c, 16)]
                p = pos_v[prow, pl.ds(16 * c, 16)]
                e.append(x + p)
            s = (e[0] + e[1]) + (e[2] + e[3])
            q = (e[0] * e[0] + e[1] * e[1]) + (e[2] * e[2] + e[3] * e[3])
            mean = _lane_sum(s) * (1.0 / EMBED_DIM)
            var = _lane_sum(q) * (1.0 / EMBED_DIM) - mean * mean + EPS
            rstd = _rsqrt16(var)
            for c in range(4):
                rows_b[row, pl.ds(16 * c, 16)] = (e[c] - mean) * rstd * gam[c] + bet[c]
            return carry

        lax.fori_loop(0, _CHUNK, row_body_v1, 0, unroll=4)
        return

        def group_body(g, carry):
            rbase = g * 16
            for rr in range(16):
                row = rbase + rr
                prow = row - jnp.where(row >= MAX_POS, MAX_POS, 0)
                e = []
                for c in range(4):
                    x = rows_b[row, pl.ds(16 * c, 16)]
                    p = pos_v[prow, pl.ds(16 * c, 16)]
                    e.append(x + p)
                    rows_b[row, pl.ds(16 * c, 16)] = e[c]
                s = (e[0] + e[1]) + (e[2] + e[3])
                q = (e[0] * e[0] + e[1] * e[1]) + (e[2] * e[2] + e[3] * e[3])
                stats_v[pl.ds(16 * rr, 16)] = s
                stats_v[pl.ds(256 + 16 * rr, 16)] = q
            accs = plsc.load_gather(stats_v, [colb])
            accq = plsc.load_gather(stats_v, [colb + 256])
            for d in range(1, 16):
                idx = colb + d
                accs = accs + plsc.load_gather(stats_v, [idx])
                accq = accq + plsc.load_gather(stats_v, [idx + 256])
            mean = accs * (1.0 / EMBED_DIM)
            var = accq * (1.0 / EMBED_DIM) - mean * mean + EPS
            rstd = _rsqrt16(var)
            mrs_v[pl.ds(0, 16)] = mean
            mrs_v[pl.ds(16, 16)] = rstd
            for rr in range(16):
                row = rbase + rr
                m = plsc.load_gather(mrs_v, [jnp.full((16,), rr, jnp.int32)])
                r = plsc.load_gather(mrs_v, [jnp.full((16,), 16 + rr, jnp.int32)])
                for c in range(4):
                    ec = rows_b[row, pl.ds(16 * c, 16)]
                    rows_b[row, pl.ds(16 * c, 16)] = (ec - m) * r * gam[c] + bet[c]
            return carry

        lax.fori_loop(0, _GROUPS, group_body, 0)

    start_idx_copy(0, 0)
    start_idx_copy(1, 1)
    wait_idx_copy(0)
    start_gather(0, 0)

    def outer_body(go, carry):
        for b in range(_NBUF):
            i = go * _NBUF + b
            nb = (b + 1) % _NBUF
            n2 = (b + 2) % _NBUF
            @pl.when(i + 2 < _NCHUNK)
            def _():
                start_idx_copy(i + 2, n2)
            @pl.when(jnp.logical_and(i + 1 < _NCHUNK, i >= 3))
            def _():
                wait_out_copy(nb)
            @pl.when(i + 1 < _NCHUNK)
            def _():
                wait_idx_copy(nb)
                start_gather(i + 1, nb)
            wait_gather(b)
            compute_chunk(row_bufs[b])
            start_out_copy(i, b)
        return carry

    lax.fori_loop(0, _NCHUNK // _NBUF, outer_body, 0)
    for b in range(_NBUF):
        wait_out_copy(b)


@functools.partial(
    pl.kernel,
    mesh=plsc.VectorSubcoreMesh(core_axis_name="c", subcore_axis_name="s"),
    out_type=jax.ShapeDtypeStruct((_ROWS, EMBED_DIM), jnp.float32),
    compiler_params=pltpu.CompilerParams(
        use_tc_tiling_on_sc=False, needs_layout_passes=False),
    scratch_types=(
        [
            pltpu.VMEM((MAX_POS, EMBED_DIM), jnp.float32),
            pltpu.VMEM((2, EMBED_DIM), jnp.float32),
            pltpu.VMEM((2 * 16 * 16,), jnp.float32),
            pltpu.VMEM((32,), jnp.float32),
        ]
        + [pltpu.VMEM((_CHUNK,), jnp.int32) for _ in range(_NBUF)]
        + [pltpu.VMEM((_CHUNK, EMBED_DIM), jnp.float32) for _ in range(_NBUF)]
        + [pltpu.SemaphoreType.DMA for _ in range(3 * _NBUF)]
    ),
)
def _embed_ln_sc(ids_hbm, tok_hbm, pos_hbm, gam_hbm, bet_hbm, out_hbm,
                 pos_v, par_v, stats_v, mrs_v,
                 i0, i1, i2, i3, r0, r1, r2, r3,
                 s0, s1, s2, s3, s4, s5, s6, s7, s8, s9, s10, s11):
    _body(ids_hbm, tok_hbm, pos_hbm, gam_hbm, bet_hbm, out_hbm,
          pos_v, par_v, stats_v, mrs_v,
          [i0, i1, i2, i3], [r0, r1, r2, r3],
          [s0, s1, s2, s3], [s4, s5, s6, s7], [s8, s9, s10, s11])


def kernel(input_ids, token_table, pos_table, gamma, beta):
    ids = input_ids.astype(jnp.int32).reshape(-1)
    out = _embed_ln_sc(ids, token_table, pos_table, gamma, beta)
    return out.reshape(BATCH, MAX_POS, EMBED_DIM)


# speedup vs baseline: 1.2572x; 1.1138x over previous
import functools

import jax
import jax.numpy as jnp
from jax import lax
from jax.experimental import pallas as pl
from jax.experimental.pallas import tpu as pltpu
from jax.experimental.pallas import tpu_sc as plsc

BATCH = 4096
MAX_POS = 200
EMBED_DIM = 64
EPS = 1e-12

_NUM_WORKERS = 32
_ROWS = BATCH * MAX_POS
_RPW = _ROWS // _NUM_WORKERS
_CHUNK = 2 * MAX_POS
_NCHUNK = _RPW // _CHUNK
_NBUF = 4
_SUBS = (128, 128, 128, 16)

_GATHER_DN = lax.GatherDimensionNumbers(
    offset_dims=(), collapsed_slice_dims=(0,), start_index_map=(0,))


def _lane_sum(v):
    lanes = lax.iota(jnp.int32, 16)
    for sh in (1, 2, 4, 8):
        idx = (lanes ^ sh)[:, None]
        v = v + lax.gather(v, idx, _GATHER_DN, slice_sizes=(1,),
                           mode=lax.GatherScatterMode.PROMISE_IN_BOUNDS)
    return v


def _rsqrt16(v):
    y = lax.bitcast_convert_type(
        jnp.full((16,), 0x5F3759DF, dtype=jnp.int32)
        - lax.shift_right_arithmetic(lax.bitcast_convert_type(v, jnp.int32), 1),
        jnp.float32,
    )
    for _ in range(3):
        y = y * (1.5 - 0.5 * v * y * y)
    return y


def _body(ids_hbm, tok_hbm, pos_hbm, gam_hbm, bet_hbm, out_hbm,
          pos_v, par_v, idx_bufs, row_bufs, isems, gsems, osems):
    wid = lax.axis_index("s") * 2 + lax.axis_index("c")
    base = wid * _RPW

    pltpu.sync_copy(pos_hbm, pos_v)
    pltpu.sync_copy(gam_hbm, par_v.at[0])
    pltpu.sync_copy(bet_hbm, par_v.at[1])
    gam = [par_v[0, pl.ds(16 * c, 16)] for c in range(4)]
    bet = [par_v[1, pl.ds(16 * c, 16)] for c in range(4)]

    def chunk_off(i):
        return pl.multiple_of(base + i * _CHUNK, 8)

    def start_idx_copy(i, b):
        pltpu.async_copy(ids_hbm.at[pl.ds(chunk_off(i), _CHUNK)],
                         idx_bufs[b], isems[b])

    def wait_idx_copy(b):
        pltpu.make_async_copy(ids_hbm.at[pl.ds(0, _CHUNK)],
                              idx_bufs[b], isems[b]).wait()

    def start_gather(i, b):
        off = 0
        for j, n in enumerate(_SUBS):
            o = pl.multiple_of(off, 8)
            pltpu.async_copy(tok_hbm.at[idx_bufs[b].at[pl.ds(o, n)]],
                             row_bufs[b].at[pl.ds(o, n)], gsems[b][j])
            off += n

    def wait_sub_gather(b, j):
        off = sum(_SUBS[:j])
        pltpu.make_async_copy(out_hbm.at[pl.ds(0, _SUBS[j])],
                              row_bufs[b].at[pl.ds(off, _SUBS[j])],
                              gsems[b][j]).wait()

    def start_out_copy(i, b):
        pltpu.async_copy(row_bufs[b],
                         out_hbm.at[pl.ds(chunk_off(i), _CHUNK)], osems[b])

    def wait_out_copy(b):
        pltpu.make_async_copy(out_hbm.at[pl.ds(0, _CHUNK)],
                              row_bufs[b], osems[b]).wait()

    def compute_rows(rows_b, start, n):
        def row_body(row, carry):
            prow = row - jnp.where(row >= MAX_POS, MAX_POS, 0)
            e = []
            for c in range(4):
                x = rows_b[row, pl.ds(16 * c, 16)]
                p = pos_v[prow, pl.ds(16 * c, 16)]
                e.append(x + p)
            s = (e[0] + e[1]) + (e[2] + e[3])
            q = (e[0] * e[0] + e[1] * e[1]) + (e[2] * e[2] + e[3] * e[3])
            mean = _lane_sum(s) * (1.0 / EMBED_DIM)
            var = _lane_sum(q) * (1.0 / EMBED_DIM) - mean * mean + EPS
            rstd = _rsqrt16(var)
            for c in range(4):
                rows_b[row, pl.ds(16 * c, 16)] = \
                    (e[c] - mean) * rstd * gam[c] + bet[c]
            return carry

        lax.fori_loop(start, start + n, row_body, 0, unroll=4)

    start_idx_copy(0, 0)
    start_idx_copy(1, 1)
    wait_idx_copy(0)
    start_gather(0, 0)

    def outer_body(go, carry):
        for b in range(_NBUF):
            i = go * _NBUF + b
            nb = (b + 1) % _NBUF
            n2 = (b + 2) % _NBUF
            @pl.when(i + 2 < _NCHUNK)
            def _():
                start_idx_copy(i + 2, n2)
            @pl.when(jnp.logical_and(i + 1 < _NCHUNK, i >= 3))
            def _():
                wait_out_copy(nb)
            @pl.when(i + 1 < _NCHUNK)
            def _():
                wait_idx_copy(nb)
                start_gather(i + 1, nb)
            off = 0
            for j, n in enumerate(_SUBS):
                wait_sub_gather(b, j)
                compute_rows(row_bufs[b], off, n)
                off += n
            start_out_copy(i, b)
        return carry

    lax.fori_loop(0, _NCHUNK // _NBUF, outer_body, 0)
    for b in range(_NBUF):
        wait_out_copy(b)


@functools.partial(
    pl.kernel,
    mesh=plsc.VectorSubcoreMesh(core_axis_name="c", subcore_axis_name="s"),
    out_type=jax.ShapeDtypeStruct((_ROWS, EMBED_DIM), jnp.float32),
    compiler_params=pltpu.CompilerParams(
        use_tc_tiling_on_sc=False, needs_layout_passes=False),
    scratch_types=(
        [
            pltpu.VMEM((MAX_POS, EMBED_DIM), jnp.float32),
            pltpu.VMEM((2, EMBED_DIM), jnp.float32),
        ]
        + [pltpu.VMEM((_CHUNK,), jnp.int32) for _ in range(_NBUF)]
        + [pltpu.VMEM((_CHUNK, EMBED_DIM), jnp.float32) for _ in range(_NBUF)]
        + [pltpu.SemaphoreType.DMA for _ in range(_NBUF * 6)]
    ),
)
def _embed_ln_sc(ids_hbm, tok_hbm, pos_hbm, gam_hbm, bet_hbm, out_hbm,
                 pos_v, par_v,
                 i0, i1, i2, i3, r0, r1, r2, r3,
                 *sems):
    isems = list(sems[0:4])
    gsems = [list(sems[4 + 4 * b:8 + 4 * b]) for b in range(4)]
    osems = list(sems[20:24])
    _body(ids_hbm, tok_hbm, pos_hbm, gam_hbm, bet_hbm, out_hbm,
          pos_v, par_v, [i0, i1, i2, i3], [r0, r1, r2, r3],
          isems, gsems, osems)


def kernel(input_ids, token_table, pos_table, gamma, beta):
    ids = input_ids.astype(jnp.int32).reshape(-1)
    out = _embed_ln_sc(ids, token_table, pos_table, gamma, beta)
    return out.reshape(BATCH, MAX_POS, EMBED_DIM)


# speedup vs baseline: 1.2618x; 1.0037x over previous
import functools

import jax
import jax.numpy as jnp
from jax import lax
from jax.experimental import pallas as pl
from jax.experimental.pallas import tpu as pltpu
from jax.experimental.pallas import tpu_sc as plsc

BATCH = 4096
MAX_POS = 200
EMBED_DIM = 64
EPS = 1e-12

_NUM_WORKERS = 32
_ROWS = BATCH * MAX_POS
_RPW = _ROWS // _NUM_WORKERS
_CHUNK = 2 * MAX_POS
_NCHUNK = _RPW // _CHUNK
_NBUF = 4
_SUBS = (128, 128, 128, 16)

_GATHER_DN = lax.GatherDimensionNumbers(
    offset_dims=(), collapsed_slice_dims=(0,), start_index_map=(0,))


def _lane_sum(v):
    lanes = lax.iota(jnp.int32, 16)
    for sh in (1, 2, 4, 8):
        idx = (lanes ^ sh)[:, None]
        v = v + lax.gather(v, idx, _GATHER_DN, slice_sizes=(1,),
                           mode=lax.GatherScatterMode.PROMISE_IN_BOUNDS)
    return v


def _rsqrt16(v):
    y = lax.bitcast_convert_type(
        jnp.full((16,), 0x5F3759DF, dtype=jnp.int32)
        - lax.shift_right_arithmetic(lax.bitcast_convert_type(v, jnp.int32), 1),
        jnp.float32,
    )
    for _ in range(3):
        y = y * (1.5 - 0.5 * v * y * y)
    return y


def _body(ids_hbm, tok_hbm, pos_hbm, gam_hbm, bet_hbm, out_hbm,
          pos_v, par_v, idx_bufs, row_bufs, isems, gsems, osems):
    wid = lax.axis_index("s") * 2 + lax.axis_index("c")
    base = wid * _RPW

    pltpu.sync_copy(pos_hbm, pos_v)
    pltpu.sync_copy(gam_hbm, par_v.at[0])
    pltpu.sync_copy(bet_hbm, par_v.at[1])
    gam = [par_v[0, pl.ds(16 * c, 16)] for c in range(4)]
    bet = [par_v[1, pl.ds(16 * c, 16)] for c in range(4)]

    def chunk_off(i):
        return pl.multiple_of(base + i * _CHUNK, 8)

    def start_idx_copy(i, b):
        pltpu.async_copy(ids_hbm.at[pl.ds(chunk_off(i), _CHUNK)],
                         idx_bufs[b], isems[b])

    def wait_idx_copy(b):
        pltpu.make_async_copy(ids_hbm.at[pl.ds(0, _CHUNK)],
                              idx_bufs[b], isems[b]).wait()

    def start_gather(i, b):
        off = 0
        for j, n in enumerate(_SUBS):
            o = pl.multiple_of(off, 8)
            pltpu.async_copy(tok_hbm.at[idx_bufs[b].at[pl.ds(o, n)]],
                             row_bufs[b].at[pl.ds(o, n)], gsems[b][j])
            off += n

    def wait_sub_gather(b, j):
        off = sum(_SUBS[:j])
        pltpu.make_async_copy(out_hbm.at[pl.ds(0, _SUBS[j])],
                              row_bufs[b].at[pl.ds(off, _SUBS[j])],
                              gsems[b][j]).wait()

    def start_out_copy(i, b):
        pltpu.async_copy(row_bufs[b],
                         out_hbm.at[pl.ds(chunk_off(i), _CHUNK)], osems[b])

    def wait_out_copy(b):
        pltpu.make_async_copy(out_hbm.at[pl.ds(0, _CHUNK)],
                              row_bufs[b], osems[b]).wait()

    def compute_rows(rows_b, start, n):
        def row_body(row, carry):
            prow = row - jnp.where(row >= MAX_POS, MAX_POS, 0)
            e = []
            for c in range(4):
                x = rows_b[row, pl.ds(16 * c, 16)]
                p = pos_v[prow, pl.ds(16 * c, 16)]
                e.append(x + p)
            s = (e[0] + e[1]) + (e[2] + e[3])
            q = (e[0] * e[0] + e[1] * e[1]) + (e[2] * e[2] + e[3] * e[3])
            mean = _lane_sum(s) * (1.0 / EMBED_DIM)
            var = _lane_sum(q) * (1.0 / EMBED_DIM) - mean * mean + EPS
            rstd = _rsqrt16(var)
            for c in range(4):
                rows_b[row, pl.ds(16 * c, 16)] = \
                    (e[c] - mean) * rstd * gam[c] + bet[c]
            return carry

        lax.fori_loop(start, start + n, row_body, 0, unroll=8)

    start_idx_copy(0, 0)
    start_idx_copy(1, 1)
    wait_idx_copy(0)
    start_gather(0, 0)

    def outer_body(go, carry):
        for b in range(_NBUF):
            i = go * _NBUF + b
            nb = (b + 1) % _NBUF
            n2 = (b + 2) % _NBUF
            @pl.when(i + 2 < _NCHUNK)
            def _():
                start_idx_copy(i + 2, n2)
            @pl.when(jnp.logical_and(i + 1 < _NCHUNK, i >= 3))
            def _():
                wait_out_copy(nb)
            @pl.when(i + 1 < _NCHUNK)
            def _():
                wait_idx_copy(nb)
                start_gather(i + 1, nb)
            off = 0
            for j, n in enumerate(_SUBS):
                wait_sub_gather(b, j)
                compute_rows(row_bufs[b], off, n)
                off += n
            start_out_copy(i, b)
        return carry

    lax.fori_loop(0, _NCHUNK // _NBUF, outer_body, 0)
    for b in range(_NBUF):
        wait_out_copy(b)


@functools.partial(
    pl.kernel,
    mesh=plsc.VectorSubcoreMesh(core_axis_name="c", subcore_axis_name="s"),
    out_type=jax.ShapeDtypeStruct((_ROWS, EMBED_DIM), jnp.float32),
    compiler_params=pltpu.CompilerParams(
        use_tc_tiling_on_sc=False, needs_layout_passes=False),
    scratch_types=(
        [
            pltpu.VMEM((MAX_POS, EMBED_DIM), jnp.float32),
            pltpu.VMEM((2, EMBED_DIM), jnp.float32),
        ]
        + [pltpu.VMEM((_CHUNK,), jnp.int32) for _ in range(_NBUF)]
        + [pltpu.VMEM((_CHUNK, EMBED_DIM), jnp.float32) for _ in range(_NBUF)]
        + [pltpu.SemaphoreType.DMA for _ in range(_NBUF * 6)]
    ),
)
def _embed_ln_sc(ids_hbm, tok_hbm, pos_hbm, gam_hbm, bet_hbm, out_hbm,
                 pos_v, par_v,
                 i0, i1, i2, i3, r0, r1, r2, r3,
                 *sems):
    isems = list(sems[0:4])
    gsems = [list(sems[4 + 4 * b:8 + 4 * b]) for b in range(4)]
    osems = list(sems[20:24])
    _body(ids_hbm, tok_hbm, pos_hbm, gam_hbm, bet_hbm, out_hbm,
          pos_v, par_v, [i0, i1, i2, i3], [r0, r1, r2, r3],
          isems, gsems, osems)


def kernel(input_ids, token_table, pos_table, gamma, beta):
    ids = input_ids.astype(jnp.int32).reshape(-1)
    out = _embed_ln_sc(ids, token_table, pos_table, gamma, beta)
    return out.reshape(BATCH, MAX_POS, EMBED_DIM)
